# baseline (speedup 1.0000x reference)
import functools

import jax
import jax.numpy as jnp
from jax.experimental import pallas as pl
from jax.experimental.pallas import tpu as pltpu


def _se_fused_kernel(x_ref, w1_ref, b1_ref, w2_ref, b2_ref, o_ref, *, inv_hw):
    partial = jnp.sum(x_ref[...].astype(jnp.float32), axis=-1)
    pooled = jnp.sum(partial, axis=-1) * inv_hw

    h = jnp.dot(pooled, w1_ref[...], preferred_element_type=jnp.float32)
    h = jnp.maximum(h + b1_ref[...], 0.0)
    z = jnp.dot(h, w2_ref[...], preferred_element_type=jnp.float32)
    s = jax.nn.sigmoid(z + b2_ref[...])

    o_ref[...] = (x_ref[...] * s[:, :, None, None]).astype(o_ref.dtype)


def kernel(x, w1, b1, w2, b2):
    N, C, H, W = x.shape
    Cr = w1.shape[1]
    itemsize = jnp.dtype(x.dtype).itemsize

    cost = pl.CostEstimate(
        flops=int(2 * N * C * H * W + 4 * N * C * Cr),
        transcendentals=int(N * C),
        bytes_accessed=int(2 * N * C * H * W * itemsize
                           + (C * Cr + Cr + Cr * C + C) * 4),
    )

    return pl.pallas_call(
        functools.partial(_se_fused_kernel, inv_hw=1.0 / (H * W)),
        out_shape=jax.ShapeDtypeStruct((N, C, H, W), x.dtype),
        grid=(N,),
        in_specs=[
            pl.BlockSpec((1, C, H, W), lambda n: (n, 0, 0, 0)),
            pl.BlockSpec((C, Cr), lambda n: (0, 0)),
            pl.BlockSpec((1, Cr), lambda n: (0, 0)),
            pl.BlockSpec((Cr, C), lambda n: (0, 0)),
            pl.BlockSpec((1, C), lambda n: (0, 0)),
        ],
        out_specs=pl.BlockSpec((1, C, H, W), lambda n: (n, 0, 0, 0)),
        compiler_params=pltpu.CompilerParams(
            dimension_semantics=("parallel",),
            vmem_limit_bytes=56 * 1024 * 1024),
        cost_estimate=cost,
    )(x, w1, b1, w2, b2)


# speedup vs baseline: 1.2540x; 1.2540x over previous
import functools

import jax
import jax.numpy as jnp
from jax.experimental import pallas as pl
from jax.experimental.pallas import tpu as pltpu


def _se_pair_kernel(x1_ref, x2_ref, w1_ref, b1_ref, w2_ref, b2_ref,
                    o1_ref, o2_ref, *, inv_hw):
    for x_ref, o_ref in ((x1_ref, o1_ref), (x2_ref, o2_ref)):
        pooled = jnp.sum(x_ref[...].astype(jnp.float32), axis=-1) * inv_hw
        h = jnp.dot(pooled, w1_ref[...], preferred_element_type=jnp.float32)
        h = jnp.maximum(h + b1_ref[...], 0.0)
        z = jnp.dot(h, w2_ref[...], preferred_element_type=jnp.float32)
        s = jax.nn.sigmoid(z + b2_ref[...])
        o_ref[...] = (x_ref[...] * s[:, :, None]).astype(o_ref.dtype)


def kernel(x, w1, b1, w2, b2):
    N, C, H, W = x.shape
    HW = H * W
    Cr = w1.shape[1]
    half = N // 2

    x1 = x[:half].reshape(half, C, HW)
    x2 = x[half:].reshape(half, C, HW)

    o1, o2 = pl.pallas_call(
        functools.partial(_se_pair_kernel, inv_hw=1.0 / HW),
        out_shape=(jax.ShapeDtypeStruct((half, C, HW), x.dtype),
                   jax.ShapeDtypeStruct((half, C, HW), x.dtype)),
        grid=(half,),
        in_specs=[
            pl.BlockSpec((1, C, HW), lambda n: (n, 0, 0)),
            pl.BlockSpec((1, C, HW), lambda n: (n, 0, 0)),
            pl.BlockSpec((C, Cr), lambda n: (0, 0)),
            pl.BlockSpec((1, Cr), lambda n: (0, 0)),
            pl.BlockSpec((Cr, C), lambda n: (0, 0)),
            pl.BlockSpec((1, C), lambda n: (0, 0)),
        ],
        out_specs=(pl.BlockSpec((1, C, HW), lambda n: (n, 0, 0)),
                   pl.BlockSpec((1, C, HW), lambda n: (n, 0, 0))),
        compiler_params=pltpu.CompilerParams(
            dimension_semantics=("parallel",),
            vmem_limit_bytes=48 * 1024 * 1024),
    )(x1, x2, w1, b1, w2, b2)

    return jnp.concatenate([o1, o2], axis=0).reshape(N, C, H, W)


# speedup vs baseline: 1.8921x; 1.5088x over previous
import functools

import jax
import jax.numpy as jnp
from jax.experimental import pallas as pl
from jax.experimental.pallas import tpu as pltpu

_DEPTH = 4
_AHEAD = 3


def _se_pipe_kernel(x_hbm, w1_ref, b1_ref, w2_ref, b2_ref, o_hbm,
                    x_buf, o_buf, in_sem, out_sem, *, inv_hw, per_core):
    base = pl.program_id(0) * per_core

    def start_in(slot, i):
        pltpu.make_async_copy(x_hbm.at[pl.ds(base + i, 1)], x_buf.at[slot],
                              in_sem.at[slot]).start()

    def wait_in(slot):
        pltpu.make_async_copy(x_buf.at[slot], x_buf.at[slot],
                              in_sem.at[slot]).wait()

    def start_out(slot, i):
        pltpu.make_async_copy(o_buf.at[slot], o_hbm.at[pl.ds(base + i, 1)],
                              out_sem.at[slot]).start()

    def wait_out(slot):
        pltpu.make_async_copy(o_buf.at[slot], o_buf.at[slot],
                              out_sem.at[slot]).wait()

    def prologue(i, _):
        @pl.when(i < per_core)
        def _():
            start_in(jax.lax.rem(i, _DEPTH), i)
        return ()

    jax.lax.fori_loop(0, _AHEAD, prologue, (), unroll=True)

    def body(i, _):
        slot = jax.lax.rem(i, _DEPTH)

        @pl.when(i + _AHEAD < per_core)
        def _():
            start_in(jax.lax.rem(i + _AHEAD, _DEPTH), i + _AHEAD)

        wait_in(slot)

        pooled = jnp.sum(x_buf[slot].astype(jnp.float32), axis=-1) * inv_hw
        h = jnp.dot(pooled, w1_ref[...], preferred_element_type=jnp.float32)
        h = jnp.maximum(h + b1_ref[...], 0.0)
        z = jnp.dot(h, w2_ref[...], preferred_element_type=jnp.float32)
        s = jax.nn.sigmoid(z + b2_ref[...])

        @pl.when(i >= _DEPTH)
        def _():
            wait_out(slot)

        o_buf[slot] = (x_buf[slot] * s[:, :, None]).astype(o_buf.dtype)
        start_out(slot, i)
        return ()

    jax.lax.fori_loop(0, per_core, body, (), unroll=False)

    n_tail = min(_DEPTH, per_core)

    def tail(k, _):
        wait_out(jax.lax.rem(max(per_core - n_tail, 0) + k, _DEPTH))
        return ()

    jax.lax.fori_loop(0, n_tail, tail, (), unroll=True)


def kernel(x, w1, b1, w2, b2):
    N, C, H, W = x.shape
    HW = H * W
    Cr = w1.shape[1]
    itemsize = jnp.dtype(x.dtype).itemsize

    x_flat = x.reshape(N, C, HW)

    n_cores = 2 if N % 2 == 0 else 1
    per_core = N // n_cores

    cost = pl.CostEstimate(
        flops=int(2 * N * C * HW + 4 * N * C * Cr),
        transcendentals=int(N * C),
        bytes_accessed=int(2 * N * C * HW * itemsize
                           + (C * Cr + Cr + Cr * C + C) * 4),
    )

    out_flat = pl.pallas_call(
        functools.partial(_se_pipe_kernel, inv_hw=1.0 / HW,
                          per_core=per_core),
        out_shape=jax.ShapeDtypeStruct((N, C, HW), x.dtype),
        grid=(n_cores,),
        in_specs=[
            pl.BlockSpec(memory_space=pl.ANY),
            pl.BlockSpec((C, Cr), lambda c: (0, 0)),
            pl.BlockSpec((1, Cr), lambda c: (0, 0)),
            pl.BlockSpec((Cr, C), lambda c: (0, 0)),
            pl.BlockSpec((1, C), lambda c: (0, 0)),
        ],
        out_specs=pl.BlockSpec(memory_space=pl.ANY),
        scratch_shapes=[
            pltpu.VMEM((_DEPTH, 1, C, HW), x.dtype),
            pltpu.VMEM((_DEPTH, 1, C, HW), x.dtype),
            pltpu.SemaphoreType.DMA((_DEPTH,)),
            pltpu.SemaphoreType.DMA((_DEPTH,)),
        ],
        compiler_params=pltpu.CompilerParams(
            dimension_semantics=("parallel",),
            vmem_limit_bytes=48 * 1024 * 1024),
        cost_estimate=cost,
    )(x_flat, w1, b1, w2, b2)

    return out_flat.reshape(N, C, H, W)
